# baseline (speedup 1.0000x reference)
import functools

import jax
import jax.numpy as jnp
from jax import lax
from jax.experimental import pallas as pl
from jax.experimental.pallas import tpu as pltpu
from jax.experimental.pallas import tpu_sc as plsc


_BLK = 512
_NC, _NS = 2, 16
_NW = _NC * _NS
_CH = 32


def _stage1a_body(nblk_b, goff, x1_ref, x2_ref, idx_ref, w_ref):
    i = pl.program_id(0) + goff
    blk = x1_ref.shape[2]
    s = x2_ref.shape[2]

    d = jnp.zeros((blk, s), jnp.float32)
    for c in range(3):
        a = x1_ref[0, c, :]
        b = x2_ref[0, c, :]
        diff = a[:, None] - b[None, :]
        d = d + diff * diff

    big = jnp.float32(3.4e38)
    iota_col = lax.broadcasted_iota(jnp.int32, (s, 1), 0).astype(jnp.float32)
    recips = []
    idxs = []
    for _ in range(3):
        vk = jnp.min(d, axis=1, keepdims=True)
        onek = d == vk
        recips.append(1.0 / (vk + 1e-8))
        idxs.append(jnp.dot(onek.astype(jnp.float32), iota_col,
                            preferred_element_type=jnp.float32))
        d = jnp.where(onek, big, d)

    norm = recips[0] + recips[1] + recips[2]
    w3 = jnp.concatenate([recips[0] / norm, recips[1] / norm,
                          recips[2] / norm], axis=1)
    boff = (i // nblk_b * s).astype(jnp.float32)
    idx3 = jnp.concatenate(idxs, axis=1) + boff
    idx_ref[...] = idx3.astype(jnp.int32)
    w_ref[...] = w3


def _sc_gather_body(f2_ref, idx_ref, wts_ref, out_ref,
                    idx_v, w_v, rows_v0, rows_v1, out_v, sem0, sem1):
    c2 = f2_ref.shape[1]
    m = out_ref.shape[0]
    rows_per_w = m // _NW
    nch = rows_per_w // _CH
    wid = lax.axis_index("s") * _NC + lax.axis_index("c")
    base_row = wid * rows_per_w
    iota = lax.broadcasted_iota(jnp.int32, (16,), 0)
    zero16 = iota * 0
    bufs = (rows_v0, rows_v1)
    sems = (sem0, sem1)

    pltpu.sync_copy(idx_ref.at[pl.ds(base_row * 3, 3 * rows_per_w)], idx_v)
    pltpu.sync_copy(wts_ref.at[pl.ds(base_row * 3, 3 * rows_per_w)], w_v)

    def start(t, p):
        pltpu.async_copy(f2_ref.at[idx_v.at[pl.ds(t * 3 * _CH, 3 * _CH)]],
                         bufs[p], sems[p])

    def combine(t, p, slot):
        rows = bufs[p]
        pltpu.make_async_copy(f2_ref.at[idx_v.at[pl.ds(0, 3 * _CH)]],
                              rows, sems[p]).wait()
        for r in range(_CH):
            wbase = 3 * (t * _CH + r)
            w0 = plsc.load_gather(w_v, [zero16 + wbase])
            w1 = plsc.load_gather(w_v, [zero16 + (wbase + 1)])
            w2 = plsc.load_gather(w_v, [zero16 + (wbase + 2)])
            ro = slot * _CH + r
            for j in range(c2 // 16):
                a = rows[3 * r, 16 * j:16 * j + 16]
                b = rows[3 * r + 1, 16 * j:16 * j + 16]
                c = rows[3 * r + 2, 16 * j:16 * j + 16]
                out_v[ro, 16 * j:16 * j + 16] = w0 * a + w1 * b + w2 * c

    start(0, 0)

    @pl.loop(0, nch, step=4)
    def group_body(t):
        start(t + 1, 1)
        combine(t, 0, 0)
        start(t + 2, 0)
        combine(t + 1, 1, 1)
        start(t + 3, 1)
        combine(t + 2, 0, 2)
        start(jnp.minimum(t + 4, nch - 1), 0)
        combine(t + 3, 1, 3)
        pltpu.sync_copy(out_v, out_ref.at[pl.ds(base_row + t * _CH, 4 * _CH)])

    pltpu.make_async_copy(f2_ref.at[idx_v.at[pl.ds(0, 3 * _CH)]],
                          rows_v0, sem0).wait()


def _stage1c_body(interp_ref, f1_ref, w1a_ref, w1b_ref, y1_ref, st_ref):
    i = pl.program_id(0)
    y1 = (jnp.dot(interp_ref[...], w1a_ref[...],
                  preferred_element_type=jnp.float32)
          + jnp.dot(f1_ref[...], w1b_ref[...],
                    preferred_element_type=jnp.float32))
    y1_ref[...] = y1

    @pl.when(i == 0)
    def _():
        st_ref[...] = jnp.zeros_like(st_ref)

    ssum = jnp.sum(y1, axis=0)
    ssq = jnp.sum(y1 * y1, axis=0)
    st_ref[...] += jnp.concatenate([ssum[None, :], ssq[None, :]], axis=0)


def _bn_coeffs(st_ref, g_ref, be_ref, m):
    mean = st_ref[0:1, :] * (1.0 / m)
    var = st_ref[1:2, :] * (1.0 / m) - mean * mean
    scale = g_ref[...] * jax.lax.rsqrt(var + 1e-5)
    shift = be_ref[...] - mean * scale
    return scale, shift


def _stage2_body(m, y1_ref, st1_ref, g1_ref, be1_ref, w2_ref,
                 y2_ref, st_ref):
    i = pl.program_id(0)
    scale, shift = _bn_coeffs(st1_ref, g1_ref, be1_ref, m)
    h = jnp.maximum(y1_ref[...] * scale + shift, 0.0)
    y2 = jnp.dot(h, w2_ref[...], preferred_element_type=jnp.float32)
    y2_ref[...] = y2

    @pl.when(i == 0)
    def _():
        st_ref[...] = jnp.zeros_like(st_ref)

    ssum = jnp.sum(y2, axis=0)
    ssq = jnp.sum(y2 * y2, axis=0)
    st_ref[...] += jnp.concatenate([ssum[None, :], ssq[None, :]], axis=0)


def _stage3_body(m, y2_ref, st2_ref, g2_ref, be2_ref, out_ref):
    scale, shift = _bn_coeffs(st2_ref, g2_ref, be2_ref, m)
    out_ref[...] = jnp.maximum(y2_ref[...] * scale + shift, 0.0)


def kernel(xyz1, xyz2, features1, features2, W1, b1, g1, be1, W2, b2, g2, be2):
    B, N, _ = xyz1.shape
    S = xyz2.shape[1]
    C1 = features1.shape[2]
    C2 = features2.shape[2]
    H1 = W1.shape[0]
    H2 = W2.shape[0]
    M = B * N
    blk = _BLK
    nblk_b = N // blk
    grid = M // blk

    x1t = jnp.transpose(xyz1, (0, 2, 1))
    x2t = jnp.transpose(xyz2, (0, 2, 1))
    f1r = features1.reshape(M, C1)
    w1a_t = jnp.transpose(W1[:, :C2])
    w1b_t = jnp.transpose(W1[:, C2:])
    w2_t = jnp.transpose(W2)

    M2 = M // 2
    grid2 = M2 // blk
    f2_flat = features2.reshape(B * S, C2)

    idxw = []
    interp = []
    for h in range(2):
        off = h * grid2
        idxw.append(pl.pallas_call(
            functools.partial(_stage1a_body, nblk_b, off),
            grid=(grid2,),
            in_specs=[
                pl.BlockSpec((1, 3, blk),
                             lambda i, o=off: ((i + o) // nblk_b, 0,
                                               (i + o) % nblk_b)),
                pl.BlockSpec((1, 3, S),
                             lambda i, o=off: ((i + o) // nblk_b, 0, 0)),
            ],
            out_specs=[
                pl.BlockSpec((blk, 3), lambda i: (i, 0)),
                pl.BlockSpec((blk, 3), lambda i: (i, 0)),
            ],
            out_shape=[
                jax.ShapeDtypeStruct((M2, 3), jnp.int32),
                jax.ShapeDtypeStruct((M2, 3), jnp.float32),
            ],
        )(x1t, x2t))
        idx3, w3 = idxw[h]
        idx_flat = idx3.reshape(M2 * 3)
        if h > 0:
            idx_flat, _ = lax.optimization_barrier((idx_flat, interp[h - 1]))
        interp.append(pl.kernel(
            _sc_gather_body,
            out_type=jax.ShapeDtypeStruct((M2, C2), jnp.float32),
            mesh=plsc.VectorSubcoreMesh(core_axis_name="c",
                                        subcore_axis_name="s",
                                        num_cores=_NC, num_subcores=_NS),
            scratch_types=[
                pltpu.VMEM((3 * (M2 // _NW),), jnp.int32),
                pltpu.VMEM((3 * (M2 // _NW),), jnp.float32),
                pltpu.VMEM((3 * _CH, C2), jnp.float32),
                pltpu.VMEM((3 * _CH, C2), jnp.float32),
                pltpu.VMEM((4 * _CH, C2), jnp.float32),
                pltpu.SemaphoreType.DMA,
                pltpu.SemaphoreType.DMA,
            ],
            compiler_params=pltpu.CompilerParams(needs_layout_passes=False),
        )(f2_flat, idx_flat, w3.reshape(M2 * 3)))

    y1s = []
    for h in range(2):
        off = h * grid2
        y1s.append(pl.pallas_call(
            _stage1c_body,
            grid=(grid2,),
            in_specs=[
                pl.BlockSpec((blk, C2), lambda i: (i, 0)),
                pl.BlockSpec((blk, C1), lambda i, o=off: (i + o, 0)),
                pl.BlockSpec((C2, H1), lambda i: (0, 0)),
                pl.BlockSpec((C1, H1), lambda i: (0, 0)),
            ],
            out_specs=[
                pl.BlockSpec((blk, H1), lambda i: (i, 0)),
                pl.BlockSpec((2, H1), lambda i: (0, 0)),
            ],
            out_shape=[
                jax.ShapeDtypeStruct((M2, H1), jnp.float32),
                jax.ShapeDtypeStruct((2, H1), jnp.float32),
            ],
        )(interp[h], f1r, w1a_t, w1b_t))

    st1 = y1s[0][1]
    for p in y1s[1:]:
        st1 = st1 + p[1]

    y2s = []
    for h in range(2):
        y2s.append(pl.pallas_call(
            functools.partial(_stage2_body, float(M)),
            grid=(grid2,),
            in_specs=[
                pl.BlockSpec((blk, H1), lambda i: (i, 0)),
                pl.BlockSpec((2, H1), lambda i: (0, 0)),
                pl.BlockSpec((1, H1), lambda i: (0, 0)),
                pl.BlockSpec((1, H1), lambda i: (0, 0)),
                pl.BlockSpec((H1, H2), lambda i: (0, 0)),
            ],
            out_specs=[
                pl.BlockSpec((blk, H2), lambda i: (i, 0)),
                pl.BlockSpec((2, H2), lambda i: (0, 0)),
            ],
            out_shape=[
                jax.ShapeDtypeStruct((M2, H2), jnp.float32),
                jax.ShapeDtypeStruct((2, H2), jnp.float32),
            ],
        )(y1s[h][0], st1, g1.reshape(1, H1), be1.reshape(1, H1), w2_t))

    st2 = y2s[0][1]
    for p in y2s[1:]:
        st2 = st2 + p[1]

    outs = []
    for h in range(2):
        outs.append(pl.pallas_call(
            functools.partial(_stage3_body, float(M)),
            grid=(grid2,),
            in_specs=[
                pl.BlockSpec((blk, H2), lambda i: (i, 0)),
                pl.BlockSpec((2, H2), lambda i: (0, 0)),
                pl.BlockSpec((1, H2), lambda i: (0, 0)),
                pl.BlockSpec((1, H2), lambda i: (0, 0)),
            ],
            out_specs=pl.BlockSpec((blk, H2), lambda i: (i, 0)),
            out_shape=jax.ShapeDtypeStruct((M2, H2), jnp.float32),
        )(y2s[h][0], st2, g2.reshape(1, H2), be2.reshape(1, H2)))

    out = jnp.concatenate(outs, axis=0)
    return out.reshape(B, N, H2)


# speedup vs baseline: 1.1194x; 1.1194x over previous
import functools

import jax
import jax.numpy as jnp
from jax import lax
from jax.experimental import pallas as pl
from jax.experimental.pallas import tpu as pltpu
from jax.experimental.pallas import tpu_sc as plsc


_BLK = 1024
_NC, _NS = 2, 16
_NW = _NC * _NS
_CH = 32


def _stage1a_body(nblk_b, goff, x1_ref, x2_ref, idx_ref, w_ref):
    i = pl.program_id(0) + goff
    blk = x1_ref.shape[2]
    s = x2_ref.shape[2]

    d = jnp.zeros((blk, s), jnp.float32)
    for c in range(3):
        a = x1_ref[0, c, :]
        b = x2_ref[0, c, :]
        diff = a[:, None] - b[None, :]
        d = d + diff * diff

    big = jnp.float32(3.4e38)
    iota_col = lax.broadcasted_iota(jnp.int32, (s, 1), 0).astype(jnp.float32)
    recips = []
    idxs = []
    for _ in range(3):
        vk = jnp.min(d, axis=1, keepdims=True)
        onek = d == vk
        recips.append(1.0 / (vk + 1e-8))
        idxs.append(jnp.dot(onek.astype(jnp.float32), iota_col,
                            preferred_element_type=jnp.float32))
        d = jnp.where(onek, big, d)

    norm = recips[0] + recips[1] + recips[2]
    w3 = jnp.concatenate([recips[0] / norm, recips[1] / norm,
                          recips[2] / norm], axis=1)
    boff = (i // nblk_b * s).astype(jnp.float32)
    idx3 = jnp.concatenate(idxs, axis=1) + boff
    idx_ref[...] = idx3.astype(jnp.int32)
    w_ref[...] = w3


def _sc_gather_body(f2_ref, idx_ref, wts_ref, out_ref,
                    idx_v, w_v, rows_v0, rows_v1, out_v, sem0, sem1):
    c2 = f2_ref.shape[1]
    m = out_ref.shape[0]
    rows_per_w = m // _NW
    nch = rows_per_w // _CH
    wid = lax.axis_index("s") * _NC + lax.axis_index("c")
    base_row = wid * rows_per_w
    iota = lax.broadcasted_iota(jnp.int32, (16,), 0)
    zero16 = iota * 0
    bufs = (rows_v0, rows_v1)
    sems = (sem0, sem1)

    pltpu.sync_copy(idx_ref.at[pl.ds(base_row * 3, 3 * rows_per_w)], idx_v)
    pltpu.sync_copy(wts_ref.at[pl.ds(base_row * 3, 3 * rows_per_w)], w_v)

    def start(t, p):
        pltpu.async_copy(f2_ref.at[idx_v.at[pl.ds(t * 3 * _CH, 3 * _CH)]],
                         bufs[p], sems[p])

    def combine(t, p, slot):
        rows = bufs[p]
        pltpu.make_async_copy(f2_ref.at[idx_v.at[pl.ds(0, 3 * _CH)]],
                              rows, sems[p]).wait()
        for r in range(_CH):
            wbase = 3 * (t * _CH + r)
            w0 = plsc.load_gather(w_v, [zero16 + wbase])
            w1 = plsc.load_gather(w_v, [zero16 + (wbase + 1)])
            w2 = plsc.load_gather(w_v, [zero16 + (wbase + 2)])
            ro = slot * _CH + r
            for j in range(c2 // 16):
                a = rows[3 * r, 16 * j:16 * j + 16]
                b = rows[3 * r + 1, 16 * j:16 * j + 16]
                c = rows[3 * r + 2, 16 * j:16 * j + 16]
                out_v[ro, 16 * j:16 * j + 16] = w0 * a + w1 * b + w2 * c

    start(0, 0)

    @pl.loop(0, nch, step=4)
    def group_body(t):
        start(t + 1, 1)
        combine(t, 0, 0)
        start(t + 2, 0)
        combine(t + 1, 1, 1)
        start(t + 3, 1)
        combine(t + 2, 0, 2)
        start(jnp.minimum(t + 4, nch - 1), 0)
        combine(t + 3, 1, 3)
        pltpu.sync_copy(out_v, out_ref.at[pl.ds(base_row + t * _CH, 4 * _CH)])

    pltpu.make_async_copy(f2_ref.at[idx_v.at[pl.ds(0, 3 * _CH)]],
                          rows_v0, sem0).wait()


def _stage1c_body(interp_ref, f1_ref, w1a_ref, w1b_ref, y1_ref, st_ref):
    i = pl.program_id(0)
    y1 = (jnp.dot(interp_ref[...], w1a_ref[...],
                  preferred_element_type=jnp.float32)
          + jnp.dot(f1_ref[...], w1b_ref[...],
                    preferred_element_type=jnp.float32))
    y1_ref[...] = y1

    @pl.when(i == 0)
    def _():
        st_ref[...] = jnp.zeros_like(st_ref)

    ssum = jnp.sum(y1, axis=0)
    ssq = jnp.sum(y1 * y1, axis=0)
    st_ref[...] += jnp.concatenate([ssum[None, :], ssq[None, :]], axis=0)


def _bn_coeffs(st_ref, g_ref, be_ref, m):
    mean = st_ref[0:1, :] * (1.0 / m)
    var = st_ref[1:2, :] * (1.0 / m) - mean * mean
    scale = g_ref[...] * jax.lax.rsqrt(var + 1e-5)
    shift = be_ref[...] - mean * scale
    return scale, shift


def _stage2_body(m, y1_ref, st1_ref, g1_ref, be1_ref, w2_ref,
                 y2_ref, st_ref):
    i = pl.program_id(0)
    scale, shift = _bn_coeffs(st1_ref, g1_ref, be1_ref, m)
    h = jnp.maximum(y1_ref[...] * scale + shift, 0.0)
    y2 = jnp.dot(h, w2_ref[...], preferred_element_type=jnp.float32)
    y2_ref[...] = y2

    @pl.when(i == 0)
    def _():
        st_ref[...] = jnp.zeros_like(st_ref)

    ssum = jnp.sum(y2, axis=0)
    ssq = jnp.sum(y2 * y2, axis=0)
    st_ref[...] += jnp.concatenate([ssum[None, :], ssq[None, :]], axis=0)


def _stage3_body(m, y2_ref, st2_ref, g2_ref, be2_ref, out_ref):
    scale, shift = _bn_coeffs(st2_ref, g2_ref, be2_ref, m)
    out_ref[...] = jnp.maximum(y2_ref[...] * scale + shift, 0.0)


def kernel(xyz1, xyz2, features1, features2, W1, b1, g1, be1, W2, b2, g2, be2):
    B, N, _ = xyz1.shape
    S = xyz2.shape[1]
    C1 = features1.shape[2]
    C2 = features2.shape[2]
    H1 = W1.shape[0]
    H2 = W2.shape[0]
    M = B * N
    blk = _BLK
    nblk_b = N // blk
    grid = M // blk

    x1t = jnp.transpose(xyz1, (0, 2, 1))
    x2t = jnp.transpose(xyz2, (0, 2, 1))
    f1r = features1.reshape(M, C1)
    w1a_t = jnp.transpose(W1[:, :C2])
    w1b_t = jnp.transpose(W1[:, C2:])
    w2_t = jnp.transpose(W2)

    M2 = M // 2
    grid2 = M2 // blk
    f2_flat = features2.reshape(B * S, C2)

    idxw = []
    interp = []
    for h in range(2):
        off = h * grid2
        idxw.append(pl.pallas_call(
            functools.partial(_stage1a_body, nblk_b, off),
            grid=(grid2,),
            in_specs=[
                pl.BlockSpec((1, 3, blk),
                             lambda i, o=off: ((i + o) // nblk_b, 0,
                                               (i + o) % nblk_b)),
                pl.BlockSpec((1, 3, S),
                             lambda i, o=off: ((i + o) // nblk_b, 0, 0)),
            ],
            out_specs=[
                pl.BlockSpec((blk, 3), lambda i: (i, 0)),
                pl.BlockSpec((blk, 3), lambda i: (i, 0)),
            ],
            out_shape=[
                jax.ShapeDtypeStruct((M2, 3), jnp.int32),
                jax.ShapeDtypeStruct((M2, 3), jnp.float32),
            ],
        )(x1t, x2t))
        idx3, w3 = idxw[h]
        idx_flat = idx3.reshape(M2 * 3)
        if h > 0:
            idx_flat, _ = lax.optimization_barrier((idx_flat, interp[h - 1]))
        interp.append(pl.kernel(
            _sc_gather_body,
            out_type=jax.ShapeDtypeStruct((M2, C2), jnp.float32),
            mesh=plsc.VectorSubcoreMesh(core_axis_name="c",
                                        subcore_axis_name="s",
                                        num_cores=_NC, num_subcores=_NS),
            scratch_types=[
                pltpu.VMEM((3 * (M2 // _NW),), jnp.int32),
                pltpu.VMEM((3 * (M2 // _NW),), jnp.float32),
                pltpu.VMEM((3 * _CH, C2), jnp.float32),
                pltpu.VMEM((3 * _CH, C2), jnp.float32),
                pltpu.VMEM((4 * _CH, C2), jnp.float32),
                pltpu.SemaphoreType.DMA,
                pltpu.SemaphoreType.DMA,
            ],
            compiler_params=pltpu.CompilerParams(needs_layout_passes=False),
        )(f2_flat, idx_flat, w3.reshape(M2 * 3)))

    y1s = []
    for h in range(2):
        off = h * grid2
        y1s.append(pl.pallas_call(
            _stage1c_body,
            grid=(grid2,),
            in_specs=[
                pl.BlockSpec((blk, C2), lambda i: (i, 0)),
                pl.BlockSpec((blk, C1), lambda i, o=off: (i + o, 0)),
                pl.BlockSpec((C2, H1), lambda i: (0, 0)),
                pl.BlockSpec((C1, H1), lambda i: (0, 0)),
            ],
            out_specs=[
                pl.BlockSpec((blk, H1), lambda i: (i, 0)),
                pl.BlockSpec((2, H1), lambda i: (0, 0)),
            ],
            out_shape=[
                jax.ShapeDtypeStruct((M2, H1), jnp.float32),
                jax.ShapeDtypeStruct((2, H1), jnp.float32),
            ],
        )(interp[h], f1r, w1a_t, w1b_t))

    st1 = y1s[0][1]
    for p in y1s[1:]:
        st1 = st1 + p[1]

    y2s = []
    for h in range(2):
        y2s.append(pl.pallas_call(
            functools.partial(_stage2_body, float(M)),
            grid=(grid2,),
            in_specs=[
                pl.BlockSpec((blk, H1), lambda i: (i, 0)),
                pl.BlockSpec((2, H1), lambda i: (0, 0)),
                pl.BlockSpec((1, H1), lambda i: (0, 0)),
                pl.BlockSpec((1, H1), lambda i: (0, 0)),
                pl.BlockSpec((H1, H2), lambda i: (0, 0)),
            ],
            out_specs=[
                pl.BlockSpec((blk, H2), lambda i: (i, 0)),
                pl.BlockSpec((2, H2), lambda i: (0, 0)),
            ],
            out_shape=[
                jax.ShapeDtypeStruct((M2, H2), jnp.float32),
                jax.ShapeDtypeStruct((2, H2), jnp.float32),
            ],
        )(y1s[h][0], st1, g1.reshape(1, H1), be1.reshape(1, H1), w2_t))

    st2 = y2s[0][1]
    for p in y2s[1:]:
        st2 = st2 + p[1]

    outs = []
    for h in range(2):
        outs.append(pl.pallas_call(
            functools.partial(_stage3_body, float(M)),
            grid=(grid2,),
            in_specs=[
                pl.BlockSpec((blk, H2), lambda i: (i, 0)),
                pl.BlockSpec((2, H2), lambda i: (0, 0)),
                pl.BlockSpec((1, H2), lambda i: (0, 0)),
                pl.BlockSpec((1, H2), lambda i: (0, 0)),
            ],
            out_specs=pl.BlockSpec((blk, H2), lambda i: (i, 0)),
            out_shape=jax.ShapeDtypeStruct((M2, H2), jnp.float32),
        )(y2s[h][0], st2, g2.reshape(1, H2), be2.reshape(1, H2)))

    out = jnp.concatenate(outs, axis=0)
    return out.reshape(B, N, H2)


# speedup vs baseline: 1.1796x; 1.0537x over previous
import functools

import jax
import jax.numpy as jnp
from jax import lax
from jax.experimental import pallas as pl
from jax.experimental.pallas import tpu as pltpu
from jax.experimental.pallas import tpu_sc as plsc


_BLK = 2048
_NC, _NS = 2, 16
_NW = _NC * _NS
_CH = 32


def _stage1a_body(nblk_b, goff, x1_ref, x2_ref, idx_ref, w_ref):
    i = pl.program_id(0) + goff
    blk = x1_ref.shape[2]
    s = x2_ref.shape[2]

    d = jnp.zeros((blk, s), jnp.float32)
    for c in range(3):
        a = x1_ref[0, c, :]
        b = x2_ref[0, c, :]
        diff = a[:, None] - b[None, :]
        d = d + diff * diff

    big = jnp.float32(3.4e38)
    iota_col = lax.broadcasted_iota(jnp.int32, (s, 1), 0).astype(jnp.float32)
    recips = []
    idxs = []
    for _ in range(3):
        vk = jnp.min(d, axis=1, keepdims=True)
        onek = d == vk
        recips.append(1.0 / (vk + 1e-8))
        idxs.append(jnp.dot(onek.astype(jnp.float32), iota_col,
                            preferred_element_type=jnp.float32))
        d = jnp.where(onek, big, d)

    norm = recips[0] + recips[1] + recips[2]
    w3 = jnp.concatenate([recips[0] / norm, recips[1] / norm,
                          recips[2] / norm], axis=1)
    boff = (i // nblk_b * s).astype(jnp.float32)
    idx3 = jnp.concatenate(idxs, axis=1) + boff
    idx_ref[...] = idx3.astype(jnp.int32)
    w_ref[...] = w3


def _sc_gather_body(f2_ref, idx_ref, wts_ref, out_ref,
                    idx_v, w_v, rows_v0, rows_v1, out_v, sem0, sem1):
    c2 = f2_ref.shape[1]
    m = out_ref.shape[0]
    rows_per_w = m // _NW
    nch = rows_per_w // _CH
    wid = lax.axis_index("s") * _NC + lax.axis_index("c")
    base_row = wid * rows_per_w
    iota = lax.broadcasted_iota(jnp.int32, (16,), 0)
    zero16 = iota * 0
    bufs = (rows_v0, rows_v1)
    sems = (sem0, sem1)

    pltpu.sync_copy(idx_ref.at[pl.ds(base_row * 3, 3 * rows_per_w)], idx_v)
    pltpu.sync_copy(wts_ref.at[pl.ds(base_row * 3, 3 * rows_per_w)], w_v)

    def start(t, p):
        pltpu.async_copy(f2_ref.at[idx_v.at[pl.ds(t * 3 * _CH, 3 * _CH)]],
                         bufs[p], sems[p])

    def combine(t, p, slot):
        rows = bufs[p]
        pltpu.make_async_copy(f2_ref.at[idx_v.at[pl.ds(0, 3 * _CH)]],
                              rows, sems[p]).wait()
        for r in range(_CH):
            wbase = 3 * (t * _CH + r)
            w0 = plsc.load_gather(w_v, [zero16 + wbase])
            w1 = plsc.load_gather(w_v, [zero16 + (wbase + 1)])
            w2 = plsc.load_gather(w_v, [zero16 + (wbase + 2)])
            ro = slot * _CH + r
            for j in range(c2 // 16):
                a = rows[3 * r, 16 * j:16 * j + 16]
                b = rows[3 * r + 1, 16 * j:16 * j + 16]
                c = rows[3 * r + 2, 16 * j:16 * j + 16]
                out_v[ro, 16 * j:16 * j + 16] = w0 * a + w1 * b + w2 * c

    start(0, 0)

    @pl.loop(0, nch, step=4)
    def group_body(t):
        start(t + 1, 1)
        combine(t, 0, 0)
        start(t + 2, 0)
        combine(t + 1, 1, 1)
        start(t + 3, 1)
        combine(t + 2, 0, 2)
        start(jnp.minimum(t + 4, nch - 1), 0)
        combine(t + 3, 1, 3)
        pltpu.sync_copy(out_v, out_ref.at[pl.ds(base_row + t * _CH, 4 * _CH)])

    pltpu.make_async_copy(f2_ref.at[idx_v.at[pl.ds(0, 3 * _CH)]],
                          rows_v0, sem0).wait()


def _stage1c_body(interp_ref, f1_ref, w1a_ref, w1b_ref, y1_ref, st_ref):
    i = pl.program_id(0)
    y1 = (jnp.dot(interp_ref[...], w1a_ref[...],
                  preferred_element_type=jnp.float32)
          + jnp.dot(f1_ref[...], w1b_ref[...],
                    preferred_element_type=jnp.float32))
    y1_ref[...] = y1

    @pl.when(i == 0)
    def _():
        st_ref[...] = jnp.zeros_like(st_ref)

    ssum = jnp.sum(y1, axis=0)
    ssq = jnp.sum(y1 * y1, axis=0)
    st_ref[...] += jnp.concatenate([ssum[None, :], ssq[None, :]], axis=0)


def _bn_coeffs(st_ref, g_ref, be_ref, m):
    mean = st_ref[0:1, :] * (1.0 / m)
    var = st_ref[1:2, :] * (1.0 / m) - mean * mean
    scale = g_ref[...] * jax.lax.rsqrt(var + 1e-5)
    shift = be_ref[...] - mean * scale
    return scale, shift


def _stage2_body(m, y1_ref, st1_ref, g1_ref, be1_ref, w2_ref,
                 y2_ref, st_ref):
    i = pl.program_id(0)
    scale, shift = _bn_coeffs(st1_ref, g1_ref, be1_ref, m)
    h = jnp.maximum(y1_ref[...] * scale + shift, 0.0)
    y2 = jnp.dot(h, w2_ref[...], preferred_element_type=jnp.float32)
    y2_ref[...] = y2

    @pl.when(i == 0)
    def _():
        st_ref[...] = jnp.zeros_like(st_ref)

    ssum = jnp.sum(y2, axis=0)
    ssq = jnp.sum(y2 * y2, axis=0)
    st_ref[...] += jnp.concatenate([ssum[None, :], ssq[None, :]], axis=0)


def _stage3_body(m, y2_ref, st2_ref, g2_ref, be2_ref, out_ref):
    scale, shift = _bn_coeffs(st2_ref, g2_ref, be2_ref, m)
    out_ref[...] = jnp.maximum(y2_ref[...] * scale + shift, 0.0)


def kernel(xyz1, xyz2, features1, features2, W1, b1, g1, be1, W2, b2, g2, be2):
    B, N, _ = xyz1.shape
    S = xyz2.shape[1]
    C1 = features1.shape[2]
    C2 = features2.shape[2]
    H1 = W1.shape[0]
    H2 = W2.shape[0]
    M = B * N
    blk = _BLK
    nblk_b = N // blk
    grid = M // blk

    x1t = jnp.transpose(xyz1, (0, 2, 1))
    x2t = jnp.transpose(xyz2, (0, 2, 1))
    f1r = features1.reshape(M, C1)
    w1a_t = jnp.transpose(W1[:, :C2])
    w1b_t = jnp.transpose(W1[:, C2:])
    w2_t = jnp.transpose(W2)

    M2 = M // 2
    grid2 = M2 // blk
    f2_flat = features2.reshape(B * S, C2)

    idxw = []
    interp = []
    for h in range(2):
        off = h * grid2
        idxw.append(pl.pallas_call(
            functools.partial(_stage1a_body, nblk_b, off),
            grid=(grid2,),
            in_specs=[
                pl.BlockSpec((1, 3, blk),
                             lambda i, o=off: ((i + o) // nblk_b, 0,
                                               (i + o) % nblk_b)),
                pl.BlockSpec((1, 3, S),
                             lambda i, o=off: ((i + o) // nblk_b, 0, 0)),
            ],
            out_specs=[
                pl.BlockSpec((blk, 3), lambda i: (i, 0)),
                pl.BlockSpec((blk, 3), lambda i: (i, 0)),
            ],
            out_shape=[
                jax.ShapeDtypeStruct((M2, 3), jnp.int32),
                jax.ShapeDtypeStruct((M2, 3), jnp.float32),
            ],
        )(x1t, x2t))
        idx3, w3 = idxw[h]
        idx_flat = idx3.reshape(M2 * 3)
        if h > 0:
            idx_flat, _ = lax.optimization_barrier((idx_flat, interp[h - 1]))
        interp.append(pl.kernel(
            _sc_gather_body,
            out_type=jax.ShapeDtypeStruct((M2, C2), jnp.float32),
            mesh=plsc.VectorSubcoreMesh(core_axis_name="c",
                                        subcore_axis_name="s",
                                        num_cores=_NC, num_subcores=_NS),
            scratch_types=[
                pltpu.VMEM((3 * (M2 // _NW),), jnp.int32),
                pltpu.VMEM((3 * (M2 // _NW),), jnp.float32),
                pltpu.VMEM((3 * _CH, C2), jnp.float32),
                pltpu.VMEM((3 * _CH, C2), jnp.float32),
                pltpu.VMEM((4 * _CH, C2), jnp.float32),
                pltpu.SemaphoreType.DMA,
                pltpu.SemaphoreType.DMA,
            ],
            compiler_params=pltpu.CompilerParams(needs_layout_passes=False),
        )(f2_flat, idx_flat, w3.reshape(M2 * 3)))

    y1s = []
    for h in range(2):
        off = h * grid2
        y1s.append(pl.pallas_call(
            _stage1c_body,
            grid=(grid2,),
            in_specs=[
                pl.BlockSpec((blk, C2), lambda i: (i, 0)),
                pl.BlockSpec((blk, C1), lambda i, o=off: (i + o, 0)),
                pl.BlockSpec((C2, H1), lambda i: (0, 0)),
                pl.BlockSpec((C1, H1), lambda i: (0, 0)),
            ],
            out_specs=[
                pl.BlockSpec((blk, H1), lambda i: (i, 0)),
                pl.BlockSpec((2, H1), lambda i: (0, 0)),
            ],
            out_shape=[
                jax.ShapeDtypeStruct((M2, H1), jnp.float32),
                jax.ShapeDtypeStruct((2, H1), jnp.float32),
            ],
        )(interp[h], f1r, w1a_t, w1b_t))

    st1 = y1s[0][1]
    for p in y1s[1:]:
        st1 = st1 + p[1]

    y2s = []
    for h in range(2):
        y2s.append(pl.pallas_call(
            functools.partial(_stage2_body, float(M)),
            grid=(grid2,),
            in_specs=[
                pl.BlockSpec((blk, H1), lambda i: (i, 0)),
                pl.BlockSpec((2, H1), lambda i: (0, 0)),
                pl.BlockSpec((1, H1), lambda i: (0, 0)),
                pl.BlockSpec((1, H1), lambda i: (0, 0)),
                pl.BlockSpec((H1, H2), lambda i: (0, 0)),
            ],
            out_specs=[
                pl.BlockSpec((blk, H2), lambda i: (i, 0)),
                pl.BlockSpec((2, H2), lambda i: (0, 0)),
            ],
            out_shape=[
                jax.ShapeDtypeStruct((M2, H2), jnp.float32),
                jax.ShapeDtypeStruct((2, H2), jnp.float32),
            ],
        )(y1s[h][0], st1, g1.reshape(1, H1), be1.reshape(1, H1), w2_t))

    st2 = y2s[0][1]
    for p in y2s[1:]:
        st2 = st2 + p[1]

    outs = []
    for h in range(2):
        outs.append(pl.pallas_call(
            functools.partial(_stage3_body, float(M)),
            grid=(grid2,),
            in_specs=[
                pl.BlockSpec((blk, H2), lambda i: (i, 0)),
                pl.BlockSpec((2, H2), lambda i: (0, 0)),
                pl.BlockSpec((1, H2), lambda i: (0, 0)),
                pl.BlockSpec((1, H2), lambda i: (0, 0)),
            ],
            out_specs=pl.BlockSpec((blk, H2), lambda i: (i, 0)),
            out_shape=jax.ShapeDtypeStruct((M2, H2), jnp.float32),
        )(y2s[h][0], st2, g2.reshape(1, H2), be2.reshape(1, H2)))

    out = jnp.concatenate(outs, axis=0)
    return out.reshape(B, N, H2)


# speedup vs baseline: 1.2156x; 1.0305x over previous
import functools

import jax
import jax.numpy as jnp
from jax import lax
from jax.experimental import pallas as pl
from jax.experimental.pallas import tpu as pltpu
from jax.experimental.pallas import tpu_sc as plsc


_BLK = 4096
_NC, _NS = 2, 16
_NW = _NC * _NS
_CH = 32


def _stage1a_body(nblk_b, goff, x1_ref, x2_ref, idx_ref, w_ref):
    i = pl.program_id(0) + goff
    blk = x1_ref.shape[2]
    s = x2_ref.shape[2]

    d = jnp.zeros((blk, s), jnp.float32)
    for c in range(3):
        a = x1_ref[0, c, :]
        b = x2_ref[0, c, :]
        diff = a[:, None] - b[None, :]
        d = d + diff * diff

    big = jnp.float32(3.4e38)
    iota_col = lax.broadcasted_iota(jnp.int32, (s, 1), 0).astype(jnp.float32)
    recips = []
    idxs = []
    for _ in range(3):
        vk = jnp.min(d, axis=1, keepdims=True)
        onek = d == vk
        recips.append(1.0 / (vk + 1e-8))
        idxs.append(jnp.dot(onek.astype(jnp.float32), iota_col,
                            preferred_element_type=jnp.float32))
        d = jnp.where(onek, big, d)

    norm = recips[0] + recips[1] + recips[2]
    w3 = jnp.concatenate([recips[0] / norm, recips[1] / norm,
                          recips[2] / norm], axis=1)
    boff = (i // nblk_b * s).astype(jnp.float32)
    idx3 = jnp.concatenate(idxs, axis=1) + boff
    idx_ref[...] = idx3.astype(jnp.int32)
    w_ref[...] = w3


def _sc_gather_body(f2_ref, idx_ref, wts_ref, out_ref,
                    idx_v, w_v, rows_v0, rows_v1, out_v, sem0, sem1):
    c2 = f2_ref.shape[1]
    m = out_ref.shape[0]
    rows_per_w = m // _NW
    nch = rows_per_w // _CH
    wid = lax.axis_index("s") * _NC + lax.axis_index("c")
    base_row = wid * rows_per_w
    iota = lax.broadcasted_iota(jnp.int32, (16,), 0)
    zero16 = iota * 0
    bufs = (rows_v0, rows_v1)
    sems = (sem0, sem1)

    pltpu.sync_copy(idx_ref.at[pl.ds(base_row * 3, 3 * rows_per_w)], idx_v)
    pltpu.sync_copy(wts_ref.at[pl.ds(base_row * 3, 3 * rows_per_w)], w_v)

    def start(t, p):
        pltpu.async_copy(f2_ref.at[idx_v.at[pl.ds(t * 3 * _CH, 3 * _CH)]],
                         bufs[p], sems[p])

    def combine(t, p, slot):
        rows = bufs[p]
        pltpu.make_async_copy(f2_ref.at[idx_v.at[pl.ds(0, 3 * _CH)]],
                              rows, sems[p]).wait()
        for r in range(_CH):
            wbase = 3 * (t * _CH + r)
            w0 = plsc.load_gather(w_v, [zero16 + wbase])
            w1 = plsc.load_gather(w_v, [zero16 + (wbase + 1)])
            w2 = plsc.load_gather(w_v, [zero16 + (wbase + 2)])
            ro = slot * _CH + r
            for j in range(c2 // 16):
                a = rows[3 * r, 16 * j:16 * j + 16]
                b = rows[3 * r + 1, 16 * j:16 * j + 16]
                c = rows[3 * r + 2, 16 * j:16 * j + 16]
                out_v[ro, 16 * j:16 * j + 16] = w0 * a + w1 * b + w2 * c

    start(0, 0)

    @pl.loop(0, nch, step=4)
    def group_body(t):
        start(t + 1, 1)
        combine(t, 0, 0)
        start(t + 2, 0)
        combine(t + 1, 1, 1)
        start(t + 3, 1)
        combine(t + 2, 0, 2)
        start(jnp.minimum(t + 4, nch - 1), 0)
        combine(t + 3, 1, 3)
        pltpu.sync_copy(out_v, out_ref.at[pl.ds(base_row + t * _CH, 4 * _CH)])

    pltpu.make_async_copy(f2_ref.at[idx_v.at[pl.ds(0, 3 * _CH)]],
                          rows_v0, sem0).wait()


def _stage1c_body(interp_ref, f1_ref, w1a_ref, w1b_ref, y1_ref, st_ref):
    i = pl.program_id(0)
    y1 = (jnp.dot(interp_ref[...], w1a_ref[...],
                  preferred_element_type=jnp.float32)
          + jnp.dot(f1_ref[...], w1b_ref[...],
                    preferred_element_type=jnp.float32))
    y1_ref[...] = y1

    @pl.when(i == 0)
    def _():
        st_ref[...] = jnp.zeros_like(st_ref)

    ssum = jnp.sum(y1, axis=0)
    ssq = jnp.sum(y1 * y1, axis=0)
    st_ref[...] += jnp.concatenate([ssum[None, :], ssq[None, :]], axis=0)


def _bn_coeffs(st_ref, g_ref, be_ref, m):
    mean = st_ref[0:1, :] * (1.0 / m)
    var = st_ref[1:2, :] * (1.0 / m) - mean * mean
    scale = g_ref[...] * jax.lax.rsqrt(var + 1e-5)
    shift = be_ref[...] - mean * scale
    return scale, shift


def _stage2_body(m, y1_ref, st1_ref, g1_ref, be1_ref, w2_ref,
                 y2_ref, st_ref):
    i = pl.program_id(0)
    scale, shift = _bn_coeffs(st1_ref, g1_ref, be1_ref, m)
    h = jnp.maximum(y1_ref[...] * scale + shift, 0.0)
    y2 = jnp.dot(h, w2_ref[...], preferred_element_type=jnp.float32)
    y2_ref[...] = y2

    @pl.when(i == 0)
    def _():
        st_ref[...] = jnp.zeros_like(st_ref)

    ssum = jnp.sum(y2, axis=0)
    ssq = jnp.sum(y2 * y2, axis=0)
    st_ref[...] += jnp.concatenate([ssum[None, :], ssq[None, :]], axis=0)


def _stage3_body(m, y2_ref, st2_ref, g2_ref, be2_ref, out_ref):
    scale, shift = _bn_coeffs(st2_ref, g2_ref, be2_ref, m)
    out_ref[...] = jnp.maximum(y2_ref[...] * scale + shift, 0.0)


def kernel(xyz1, xyz2, features1, features2, W1, b1, g1, be1, W2, b2, g2, be2):
    B, N, _ = xyz1.shape
    S = xyz2.shape[1]
    C1 = features1.shape[2]
    C2 = features2.shape[2]
    H1 = W1.shape[0]
    H2 = W2.shape[0]
    M = B * N
    blk = _BLK
    nblk_b = N // blk
    grid = M // blk

    x1t = jnp.transpose(xyz1, (0, 2, 1))
    x2t = jnp.transpose(xyz2, (0, 2, 1))
    f1r = features1.reshape(M, C1)
    w1a_t = jnp.transpose(W1[:, :C2])
    w1b_t = jnp.transpose(W1[:, C2:])
    w2_t = jnp.transpose(W2)

    M2 = M // 2
    grid2 = M2 // blk
    f2_flat = features2.reshape(B * S, C2)

    idxw = []
    interp = []
    for h in range(2):
        off = h * grid2
        idxw.append(pl.pallas_call(
            functools.partial(_stage1a_body, nblk_b, off),
            grid=(grid2,),
            in_specs=[
                pl.BlockSpec((1, 3, blk),
                             lambda i, o=off: ((i + o) // nblk_b, 0,
                                               (i + o) % nblk_b)),
                pl.BlockSpec((1, 3, S),
                             lambda i, o=off: ((i + o) // nblk_b, 0, 0)),
            ],
            out_specs=[
                pl.BlockSpec((blk, 3), lambda i: (i, 0)),
                pl.BlockSpec((blk, 3), lambda i: (i, 0)),
            ],
            out_shape=[
                jax.ShapeDtypeStruct((M2, 3), jnp.int32),
                jax.ShapeDtypeStruct((M2, 3), jnp.float32),
            ],
        )(x1t, x2t))
        idx3, w3 = idxw[h]
        idx_flat = idx3.reshape(M2 * 3)
        if h > 0:
            idx_flat, _ = lax.optimization_barrier((idx_flat, interp[h - 1]))
        interp.append(pl.kernel(
            _sc_gather_body,
            out_type=jax.ShapeDtypeStruct((M2, C2), jnp.float32),
            mesh=plsc.VectorSubcoreMesh(core_axis_name="c",
                                        subcore_axis_name="s",
                                        num_cores=_NC, num_subcores=_NS),
            scratch_types=[
                pltpu.VMEM((3 * (M2 // _NW),), jnp.int32),
                pltpu.VMEM((3 * (M2 // _NW),), jnp.float32),
                pltpu.VMEM((3 * _CH, C2), jnp.float32),
                pltpu.VMEM((3 * _CH, C2), jnp.float32),
                pltpu.VMEM((4 * _CH, C2), jnp.float32),
                pltpu.SemaphoreType.DMA,
                pltpu.SemaphoreType.DMA,
            ],
            compiler_params=pltpu.CompilerParams(needs_layout_passes=False),
        )(f2_flat, idx_flat, w3.reshape(M2 * 3)))

    y1s = []
    for h in range(2):
        off = h * grid2
        y1s.append(pl.pallas_call(
            _stage1c_body,
            grid=(grid2,),
            in_specs=[
                pl.BlockSpec((blk, C2), lambda i: (i, 0)),
                pl.BlockSpec((blk, C1), lambda i, o=off: (i + o, 0)),
                pl.BlockSpec((C2, H1), lambda i: (0, 0)),
                pl.BlockSpec((C1, H1), lambda i: (0, 0)),
            ],
            out_specs=[
                pl.BlockSpec((blk, H1), lambda i: (i, 0)),
                pl.BlockSpec((2, H1), lambda i: (0, 0)),
            ],
            out_shape=[
                jax.ShapeDtypeStruct((M2, H1), jnp.float32),
                jax.ShapeDtypeStruct((2, H1), jnp.float32),
            ],
        )(interp[h], f1r, w1a_t, w1b_t))

    st1 = y1s[0][1]
    for p in y1s[1:]:
        st1 = st1 + p[1]

    y2s = []
    for h in range(2):
        y2s.append(pl.pallas_call(
            functools.partial(_stage2_body, float(M)),
            grid=(grid2,),
            in_specs=[
                pl.BlockSpec((blk, H1), lambda i: (i, 0)),
                pl.BlockSpec((2, H1), lambda i: (0, 0)),
                pl.BlockSpec((1, H1), lambda i: (0, 0)),
                pl.BlockSpec((1, H1), lambda i: (0, 0)),
                pl.BlockSpec((H1, H2), lambda i: (0, 0)),
            ],
            out_specs=[
                pl.BlockSpec((blk, H2), lambda i: (i, 0)),
                pl.BlockSpec((2, H2), lambda i: (0, 0)),
            ],
            out_shape=[
                jax.ShapeDtypeStruct((M2, H2), jnp.float32),
                jax.ShapeDtypeStruct((2, H2), jnp.float32),
            ],
        )(y1s[h][0], st1, g1.reshape(1, H1), be1.reshape(1, H1), w2_t))

    st2 = y2s[0][1]
    for p in y2s[1:]:
        st2 = st2 + p[1]

    outs = []
    for h in range(2):
        outs.append(pl.pallas_call(
            functools.partial(_stage3_body, float(M)),
            grid=(grid2,),
            in_specs=[
                pl.BlockSpec((blk, H2), lambda i: (i, 0)),
                pl.BlockSpec((2, H2), lambda i: (0, 0)),
                pl.BlockSpec((1, H2), lambda i: (0, 0)),
                pl.BlockSpec((1, H2), lambda i: (0, 0)),
            ],
            out_specs=pl.BlockSpec((blk, H2), lambda i: (i, 0)),
            out_shape=jax.ShapeDtypeStruct((M2, H2), jnp.float32),
        )(y2s[h][0], st2, g2.reshape(1, H2), be2.reshape(1, H2)))

    out = jnp.concatenate(outs, axis=0)
    return out.reshape(B, N, H2)
